# Initial kernel scaffold; baseline (speedup 1.0000x reference)
import jax
import jax.numpy as jnp
from jax.experimental import pallas as pl


def kernel(inputs):
    raise NotImplementedError("write your pallas kernel here")


# speedup vs baseline: 1.6120x; 1.6120x over previous
import jax
import jax.numpy as jnp
from jax import lax
from jax.experimental import pallas as pl


_BLK = 1024


def _body(x_ref, o_ref):
    x = x_ref[...]
    m = jnp.max(x, axis=-1, keepdims=True)
    col = lax.broadcasted_iota(jnp.int32, x.shape, 1)
    idx = jnp.min(jnp.where(x == m, col, x.shape[-1]), axis=-1, keepdims=True)
    o_ref[...] = (col == idx).astype(o_ref.dtype)


def kernel(inputs):
    n, c = inputs.shape
    return pl.pallas_call(
        _body,
        grid=(n // _BLK,),
        in_specs=[pl.BlockSpec((_BLK, c), lambda i: (i, 0))],
        out_specs=pl.BlockSpec((_BLK, c), lambda i: (i, 0)),
        out_shape=jax.ShapeDtypeStruct((n, c), inputs.dtype),
    )(inputs)
